# baseline (speedup 1.0000x reference)
import functools

import jax
import jax.numpy as jnp
from jax import lax
from jax.experimental import pallas as pl
from jax.experimental.pallas import tpu as pltpu
from jax.experimental.pallas import tpu_sc as plsc

_VOCAB = 1000
_B = 1024
_L = 50
_LP = 56
_NJ = 8
_NW = 32
_BPW = _B // _NW
_NCHUNK = _BPW * _NJ // 2
_CK = 128

_mesh = plsc.VectorSubcoreMesh(core_axis_name="c", subcore_axis_name="s")


@functools.partial(
    pl.kernel,
    mesh=_mesh,
    out_type=[
        jax.ShapeDtypeStruct((_B, _L, _VOCAB), jnp.float32),
        jax.ShapeDtypeStruct((_B, _LP, 128), jnp.float32),
    ],
    compiler_params=pltpu.CompilerParams(use_tc_tiling_on_sc=False),
    scratch_types=[
        pltpu.VMEM((_NCHUNK * _CK,), jnp.int32),
        pltpu.VMEM((2, _CK, 128), jnp.float32),
        pltpu.SemaphoreType.DMA,
        pltpu.SemaphoreType.DMA,
    ],
)
def _embed(e_hbm, s_hbm, out_hbm, e7_hbm, idx_v, rows_v, sem0, sem1):
    cid = lax.axis_index("c")
    sid = lax.axis_index("s")
    wid = sid * 2 + cid
    b0 = wid * _BPW

    pltpu.sync_copy(e_hbm.at[pl.ds(wid * _NCHUNK * _CK, _NCHUNK * _CK)], idx_v)

    def gather(q, buf):
        pltpu.async_copy(
            s_hbm.at[idx_v.at[pl.ds(q * _CK, _CK)]],
            rows_v.at[buf],
            sem0 if buf == 0 else sem1,
        )

    def gwait(buf):
        pltpu.make_async_copy(
            s_hbm.at[idx_v.at[pl.ds(0, _CK)]],
            rows_v.at[buf],
            sem0 if buf == 0 else sem1,
        ).wait()

    def writeback(q, buf):
        b = b0 + q // 4
        jp = q % 4
        j0 = 2 * jp

        @pl.when(jp == 3)
        def _():
            pltpu.sync_copy(rows_v.at[buf, pl.ds(64, _LP)], e7_hbm.at[b])

    gather(0, 0)

    def body(p, carry):
        q0 = p * 2
        gwait(0)
        gather(q0 + 1, 1)
        writeback(q0, 0)

        gwait(1)

        @pl.when(q0 + 2 < _NCHUNK)
        def _():
            gather(q0 + 2, 0)

        writeback(q0 + 1, 1)
        return carry

    lax.fori_loop(0, _NCHUNK // 2, body, 0)


def kernel(idx, targets, token_embedding_table):
    del targets
    s = jnp.pad(token_embedding_table, ((0, 0), (0, 24))).reshape(_VOCAB * 8, 128)
    idxp = jnp.pad(idx.astype(jnp.int32), ((0, 0), (0, _LP - _L)))
    base8 = idxp * 8
    jj = jnp.arange(_NJ, dtype=jnp.int32)
    ek = base8[:, None, :] + jj[None, :, None]
    ek = jnp.pad(ek, ((0, 0), (0, 0), (0, 64 - _LP)))
    e = ek.reshape(_B, 4, _CK).reshape(-1)
    main, e7 = _embed(e, s)
    return jax.lax.dynamic_update_slice(main, e7[:, :_L, :104], (0, 0, 896))


# speedup vs baseline: 4.6404x; 4.6404x over previous
import functools

import jax
import jax.numpy as jnp
from jax import lax
from jax.experimental import pallas as pl
from jax.experimental.pallas import tpu as pltpu
from jax.experimental.pallas import tpu_sc as plsc

_VOCAB = 1000
_B = 1024
_L = 50
_LP = 56
_CK = 128
_NW = 32
_BPW = _B // _NW

_mesh = plsc.VectorSubcoreMesh(core_axis_name="c", subcore_axis_name="s")


@functools.partial(
    pl.kernel,
    mesh=_mesh,
    out_type=[
        jax.ShapeDtypeStruct((_B, _L, _VOCAB), jnp.float32),
        jax.ShapeDtypeStruct((_B, _LP, 128), jnp.float32),
    ],
    scratch_types=[
        pltpu.VMEM((_BPW * _CK,), jnp.int32),
        pltpu.VMEM((2, _LP, 1024), jnp.float32),
        pltpu.SemaphoreType.DMA,
        pltpu.SemaphoreType.DMA,
        pltpu.SemaphoreType.DMA,
        pltpu.SemaphoreType.DMA,
    ],
)
def _embed(e_hbm, s_hbm, out_hbm, e7_hbm, idx_v, rows_v, gs0, gs1, ws0, ws1):
    cid = lax.axis_index("c")
    sid = lax.axis_index("s")
    wid = sid * 2 + cid
    b0 = wid * _BPW

    pltpu.sync_copy(e_hbm.at[pl.ds(wid * _BPW * _CK, _BPW * _CK)], idx_v)

    def gather(q, buf):
        pltpu.async_copy(
            s_hbm.at[idx_v.at[pl.ds(q * _CK, _LP)]],
            rows_v.at[buf],
            gs0 if buf == 0 else gs1,
        )

    def gwait(buf):
        pltpu.make_async_copy(
            s_hbm.at[idx_v.at[pl.ds(0, _LP)]],
            rows_v.at[buf],
            gs0 if buf == 0 else gs1,
        ).wait()

    def _write_list(q, buf, drain):
        b = b0 + q
        ws = ws0 if buf == 0 else ws1
        for j in range(7):
            pairs = [
                (
                    rows_v.at[buf, pl.ds(0, 48), pl.ds(j * 128, 128)],
                    out_hbm.at[b, pl.ds(0, 48), pl.ds(j * 128, 128)],
                ),
                (
                    rows_v.at[buf, pl.ds(48, 2), pl.ds(j * 128, 128)],
                    out_hbm.at[b, pl.ds(48, 2), pl.ds(j * 128, 128)],
                ),
            ]
            for src, dst in pairs:
                if drain:
                    pltpu.make_async_copy(src, dst, ws).wait()
                else:
                    pltpu.async_copy(src, dst, ws)
        src = rows_v.at[buf, pl.ds(0, _LP), pl.ds(896, 128)]
        dst = e7_hbm.at[b]
        if drain:
            pltpu.make_async_copy(src, dst, ws).wait()
        else:
            pltpu.async_copy(src, dst, ws)

    gather(0, 0)

    def body(p, carry):
        q0 = p * 2
        gwait(0)

        @pl.when(p > 0)
        def _():
            _write_list(q0 - 1, 1, drain=True)

        gather(q0 + 1, 1)
        _write_list(q0, 0, drain=False)

        gwait(1)
        _write_list(q0, 0, drain=True)

        @pl.when(q0 + 2 < _BPW)
        def _():
            gather(q0 + 2, 0)

        _write_list(q0 + 1, 1, drain=False)
        return carry

    lax.fori_loop(0, _BPW // 2, body, 0)
    _write_list(_BPW - 1, 1, drain=True)


def kernel(idx, targets, token_embedding_table):
    del targets
    s = jnp.pad(token_embedding_table, ((0, 0), (0, 24)))
    idxp = jnp.pad(idx.astype(jnp.int32), ((0, 0), (0, _LP - _L)))
    e = jnp.pad(idxp, ((0, 0), (0, _CK - _LP))).reshape(-1)
    main, e7 = _embed(e, s)
    return jax.lax.dynamic_update_slice(main, e7[:, :_L, :104], (0, 0, 896))
